# Initial kernel scaffold; baseline (speedup 1.0000x reference)
import jax
import jax.numpy as jnp
from jax.experimental import pallas as pl


def kernel(node_features_in, edges_indexes, edge_features_in, params):
    raise NotImplementedError("write your pallas kernel here")


# speedup vs baseline: 1.9907x; 1.9907x over previous
import functools

import jax
import jax.numpy as jnp
from jax import lax
from jax.experimental import pallas as pl
from jax.experimental.pallas import tpu as pltpu
from jax.experimental.pallas import tpu_sc as plsc

N_NODES = 10000
N_EDGES = 320000
D_LAT = 128

NC = 2
NS = 16
NW = NC * NS
EPW = N_EDGES // NW
CH = 128
N_FULL = EPW // CH
TAIL = EPW - N_FULL * CH
ACC_ROWS = 10240
NPS = ACC_ROWS // NS

_HIGH = jax.lax.Precision.HIGHEST


def _dot(a, b):
    return jnp.dot(a, b, precision=_HIGH, preferred_element_type=jnp.float32)


_sc_mesh = plsc.VectorSubcoreMesh(core_axis_name="c", subcore_axis_name="s")


@functools.partial(
    pl.kernel,
    out_type=[
        jax.ShapeDtypeStruct((N_EDGES, D_LAT), jnp.float32),
        jax.ShapeDtypeStruct((N_EDGES, D_LAT), jnp.float32),
    ],
    mesh=_sc_mesh,
    scratch_types=[
        pltpu.VMEM((CH,), jnp.int32),
        pltpu.VMEM((CH,), jnp.int32),
        pltpu.VMEM((CH, D_LAT), jnp.float32),
        pltpu.VMEM((CH, D_LAT), jnp.float32),
        pltpu.VMEM((TAIL,), jnp.int32),
        pltpu.VMEM((TAIL,), jnp.int32),
        pltpu.VMEM((TAIL, D_LAT), jnp.float32),
        pltpu.VMEM((TAIL, D_LAT), jnp.float32),
        pltpu.SemaphoreType.DMA,
        pltpu.SemaphoreType.DMA,
    ],
)
def _sc_gather(vs_hbm, vd_hbm, src_hbm, dst_hbm, gs_hbm, gd_hbm,
               si_v, di_v, rs_v, rd_v, si_t, di_t, rs_t, rd_t, sem_a, sem_b):
    wid = lax.axis_index("s") * NC + lax.axis_index("c")
    base = wid * EPW

    @pl.loop(0, N_FULL)
    def _(ci):
        off = base + ci * CH
        pltpu.sync_copy(src_hbm.at[pl.ds(off, CH)], si_v)
        pltpu.sync_copy(dst_hbm.at[pl.ds(off, CH)], di_v)
        a = pltpu.async_copy(vs_hbm.at[si_v], rs_v, sem_a)
        b = pltpu.async_copy(vd_hbm.at[di_v], rd_v, sem_b)
        a.wait()
        b.wait()
        pltpu.sync_copy(rs_v, gs_hbm.at[pl.ds(off, CH)])
        pltpu.sync_copy(rd_v, gd_hbm.at[pl.ds(off, CH)])

    off = base + N_FULL * CH
    pltpu.sync_copy(src_hbm.at[pl.ds(off, TAIL)], si_t)
    pltpu.sync_copy(dst_hbm.at[pl.ds(off, TAIL)], di_t)
    a = pltpu.async_copy(vs_hbm.at[si_t], rs_t, sem_a)
    b = pltpu.async_copy(vd_hbm.at[di_t], rd_t, sem_b)
    a.wait()
    b.wait()
    pltpu.sync_copy(rs_t, gs_hbm.at[pl.ds(off, TAIL)])
    pltpu.sync_copy(rd_t, gd_hbm.at[pl.ds(off, TAIL)])


@functools.partial(
    pl.kernel,
    out_type=jax.ShapeDtypeStruct((NC, ACC_ROWS, D_LAT), jnp.float32),
    mesh=_sc_mesh,
    scratch_types=[
        pltpu.VMEM((CH,), jnp.int32),
        pltpu.VMEM((CH, D_LAT), jnp.float32),
        pltpu.VMEM((TAIL,), jnp.int32),
        pltpu.VMEM((TAIL, D_LAT), jnp.float32),
        pltpu.VMEM_SHARED((ACC_ROWS, D_LAT), jnp.float32),
    ],
)
def _sc_scatter(e_hbm, dst_hbm, zeros_hbm, out_hbm,
                di_v, rows_v, di_t, rows_t, acc):
    c = lax.axis_index("c")
    s = lax.axis_index("s")
    wid = s * NC + c
    base = wid * EPW

    pltpu.sync_copy(zeros_hbm.at[pl.ds(s * NPS, NPS)], acc.at[pl.ds(s * NPS, NPS)])
    plsc.subcore_barrier()

    @pl.loop(0, N_FULL)
    def _(ci):
        off = base + ci * CH
        pltpu.sync_copy(dst_hbm.at[pl.ds(off, CH)], di_v)
        pltpu.sync_copy(e_hbm.at[pl.ds(off, CH)], rows_v)
        pltpu.sync_copy(rows_v, acc.at[di_v], add=True)

    off = base + N_FULL * CH
    pltpu.sync_copy(dst_hbm.at[pl.ds(off, TAIL)], di_t)
    pltpu.sync_copy(e_hbm.at[pl.ds(off, TAIL)], rows_t)
    pltpu.sync_copy(rows_t, acc.at[di_t], add=True)

    plsc.subcore_barrier()
    pltpu.sync_copy(acc.at[pl.ds(s * NPS, NPS)], out_hbm.at[c].at[pl.ds(s * NPS, NPS)])


_R_NODE = 2000
_R_EDGE = 2000


def _wspec(r, c):
    return pl.BlockSpec((r, c), lambda i: (0, 0))


def _rspec(r, c):
    return pl.BlockSpec((r, c), lambda i: (i, 0))


def _node_encode_body(x_ref, w1, b1, w2, b2, ws, wd, v_ref, vs_ref, vd_ref):
    h = _dot(x_ref[...], w1[...]) + b1[...]
    v = _dot(h, w2[...]) + b2[...]
    v_ref[...] = v
    vs_ref[...] = _dot(v, ws[...])
    vd_ref[...] = _dot(v, wd[...])


def _edge_step1_body(ef_ref, gs_ref, gd_ref, we1, be1, we2, be2,
                     w1e, b1, w2, b2, out_ref):
    e0 = _dot(ef_ref[...], we1[...]) + be1[...]
    e0 = _dot(e0, we2[...]) + be2[...]
    h = _dot(e0, w1e[...]) + gs_ref[...] + gd_ref[...] + b1[...]
    out_ref[...] = e0 + _dot(h, w2[...]) + b2[...]


def _edge_step2_body(e_ref, gs_ref, gd_ref, w1e, b1, w2, b2, out_ref):
    h = _dot(e_ref[...], w1e[...]) + gs_ref[...] + gd_ref[...] + b1[...]
    out_ref[...] = e_ref[...] + _dot(h, w2[...]) + b2[...]


def _node_update_body(v_ref, p0_ref, p1_ref, w1v, w1a, b1, w2, b2, ws, wd,
                      v1_ref, vs_ref, vd_ref):
    agg = p0_ref[...] + p1_ref[...]
    h = _dot(v_ref[...], w1v[...]) + _dot(agg, w1a[...]) + b1[...]
    v1 = v_ref[...] + _dot(h, w2[...]) + b2[...]
    v1_ref[...] = v1
    vs_ref[...] = _dot(v1, ws[...])
    vd_ref[...] = _dot(v1, wd[...])


def _node_final_body(v_ref, p0_ref, p1_ref, w1v, w1a, b1, w2, b2,
                     d1, db1, d2, db2, out_ref):
    agg = p0_ref[...] + p1_ref[...]
    h = _dot(v_ref[...], w1v[...]) + _dot(agg, w1a[...]) + b1[...]
    v2 = v_ref[...] + _dot(h, w2[...]) + b2[...]
    o = _dot(v2, d1[...]) + db1[...]
    out_ref[...] = _dot(o, d2[...]) + db2[...]


def _tc_call(body, grid, in_specs, out_specs, out_shapes, *args):
    return pl.pallas_call(
        body,
        grid=(grid,),
        in_specs=in_specs,
        out_specs=out_specs,
        out_shape=out_shapes,
        compiler_params=pltpu.CompilerParams(
            dimension_semantics=("arbitrary",)),
    )(*args)


def kernel(node_features_in, edges_indexes, edge_features_in, params):
    f32 = jnp.float32
    src = edges_indexes[0]
    dst = edges_indexes[1]

    def _wb(layer):
        return layer["W"], layer["b"].reshape(1, -1)

    enW1, enb1 = _wb(params["enc_node"][0])
    enW2, enb2 = _wb(params["enc_node"][1])
    eeW1, eeb1 = _wb(params["enc_edge"][0])
    eeW2, eeb2 = _wb(params["enc_edge"][1])
    dW1, db1 = _wb(params["dec"][0])
    dW2, db2 = _wb(params["dec"][1])

    steps = []
    for t in range(2):
        pe = params["proc"][t]["edge"]
        pn = params["proc"][t]["node"]
        W1, b1 = _wb(pe[0])
        W2, b2 = _wb(pe[1])
        nW1, nb1 = _wb(pn[0])
        nW2, nb2 = _wb(pn[1])
        steps.append(dict(
            W1e=W1[:D_LAT], W1s=W1[D_LAT:2 * D_LAT], W1d=W1[2 * D_LAT:],
            b1=b1, W2=W2, b2=b2,
            nW1v=nW1[:D_LAT], nW1a=nW1[D_LAT:], nb1=nb1, nW2=nW2, nb2=nb2,
        ))

    zeros_nodes = jnp.zeros((ACC_ROWS, D_LAT), f32)

    nb = N_NODES // _R_NODE
    w128 = _wspec(D_LAT, D_LAT)
    bia = _wspec(1, D_LAT)
    nrow = _rspec(_R_NODE, D_LAT)
    nshape = jax.ShapeDtypeStruct((N_NODES, D_LAT), f32)

    v0, vs1, vd1 = _tc_call(
        _node_encode_body, nb,
        [nrow, w128, bia, w128, bia, w128, w128],
        [nrow, nrow, nrow], [nshape, nshape, nshape],
        node_features_in, enW1, enb1, enW2, enb2,
        steps[0]["W1s"], steps[0]["W1d"])

    gs1, gd1 = _sc_gather(vs1, vd1, src, dst)

    eb = N_EDGES // _R_EDGE
    erow = _rspec(_R_EDGE, D_LAT)
    eshape = jax.ShapeDtypeStruct((N_EDGES, D_LAT), f32)
    e1 = _tc_call(
        _edge_step1_body, eb,
        [_rspec(_R_EDGE, 16), erow, erow,
         _wspec(16, D_LAT), bia, w128, bia, w128, bia, w128, bia],
        erow, eshape,
        edge_features_in, gs1, gd1, eeW1, eeb1, eeW2, eeb2,
        steps[0]["W1e"], steps[0]["b1"], steps[0]["W2"], steps[0]["b2"])

    part1 = _sc_scatter(e1, dst, zeros_nodes)

    v1, vs2, vd2 = _tc_call(
        _node_update_body, nb,
        [nrow, nrow, nrow, w128, w128, bia, w128, bia, w128, w128],
        [nrow, nrow, nrow], [nshape, nshape, nshape],
        v0, part1[0, :N_NODES], part1[1, :N_NODES],
        steps[0]["nW1v"], steps[0]["nW1a"], steps[0]["nb1"],
        steps[0]["nW2"], steps[0]["nb2"],
        steps[1]["W1s"], steps[1]["W1d"])

    gs2, gd2 = _sc_gather(vs2, vd2, src, dst)

    e2 = _tc_call(
        _edge_step2_body, eb,
        [erow, erow, erow, w128, bia, w128, bia],
        erow, eshape,
        e1, gs2, gd2,
        steps[1]["W1e"], steps[1]["b1"], steps[1]["W2"], steps[1]["b2"])

    part2 = _sc_scatter(e2, dst, zeros_nodes)

    out = _tc_call(
        _node_final_body, nb,
        [nrow, nrow, nrow, w128, w128, bia, w128, bia,
         w128, bia, w128, bia],
        nrow, nshape,
        v1, part2[0, :N_NODES], part2[1, :N_NODES],
        steps[1]["nW1v"], steps[1]["nW1a"], steps[1]["nb1"],
        steps[1]["nW2"], steps[1]["nb2"],
        dW1, db1, dW2, db2)

    return out


# speedup vs baseline: 2.5230x; 1.2674x over previous
import functools

import jax
import jax.numpy as jnp
from jax import lax
from jax.experimental import pallas as pl
from jax.experimental.pallas import tpu as pltpu
from jax.experimental.pallas import tpu_sc as plsc

N_NODES = 10000
N_EDGES = 320000
D_LAT = 128

NC = 2
NS = 16
NW = NC * NS
EPW = N_EDGES // NW
CH = 128
N_FULL = EPW // CH
TAIL = EPW - N_FULL * CH
ACC_ROWS = 10240
NPS = ACC_ROWS // NS

_HIGH = jax.lax.Precision.HIGHEST


def _dot(a, b):
    return jnp.dot(a, b, precision=_HIGH, preferred_element_type=jnp.float32)


_sc_mesh = plsc.VectorSubcoreMesh(core_axis_name="c", subcore_axis_name="s")

_TSL = ACC_ROWS // NS
N_EPAD = 327680
EPC = N_EPAD // NS
NF2 = EPC // CH


@functools.partial(
    pl.kernel,
    out_type=jax.ShapeDtypeStruct((NC, N_EPAD, D_LAT), jnp.float32),
    mesh=_sc_mesh,
    scratch_types=[
        pltpu.VMEM((CH,), jnp.int32),
        pltpu.VMEM((CH,), jnp.int32),
        pltpu.VMEM((CH, D_LAT), jnp.float32),
        pltpu.VMEM((CH, D_LAT), jnp.float32),
        pltpu.VMEM_SHARED((ACC_ROWS, D_LAT), jnp.float32),
    ] + [pltpu.SemaphoreType.DMA] * 6,
)
def _sc_gather(vs_hbm, vd_hbm, idx_hbm, g_hbm,
               i0, i1, r0, r1, tab_sh,
               sa0, sa1, sb0, sb1, sc0, sc1):
    c = lax.axis_index("c")
    s = lax.axis_index("s")
    base = s * EPC

    sl = pl.ds(s * _TSL, _TSL)

    @pl.when(c == 0)
    def _():
        pltpu.sync_copy(vs_hbm.at[sl], tab_sh.at[sl])

    @pl.when(c == 1)
    def _():
        pltpu.sync_copy(vd_hbm.at[sl], tab_sh.at[sl])

    plsc.subcore_barrier()

    idx = (i0, i1)
    rows = (r0, r1)
    sa = (sa0, sa1)
    sb = (sb0, sb1)
    sc = (sc0, sc1)

    def off(ci):
        return pl.ds(base + ci * CH, CH)

    for b in (0, 1):
        pltpu.async_copy(idx_hbm.at[c].at[off(b)], idx[b], sa[b])

    @pl.loop(0, NF2 // 2)
    def _(g):
        for b in (0, 1):
            ci = g * 2 + b
            pltpu.make_async_copy(idx_hbm.at[c].at[off(ci)], idx[b], sa[b]).wait()

            @pl.when(ci >= 2)
            def _():
                pltpu.make_async_copy(rows[b], g_hbm.at[c].at[off(ci - 2)], sc[b]).wait()

            pltpu.async_copy(tab_sh.at[idx[b]], rows[b], sb[b])
            pltpu.make_async_copy(tab_sh.at[idx[b]], rows[b], sb[b]).wait()
            pltpu.async_copy(rows[b], g_hbm.at[c].at[off(ci)], sc[b])

            @pl.when(ci < NF2 - 2)
            def _():
                pltpu.async_copy(idx_hbm.at[c].at[off(ci + 2)], idx[b], sa[b])

    for b in (0, 1):
        ci = NF2 - 2 + b
        pltpu.make_async_copy(rows[b], g_hbm.at[c].at[off(ci)], sc[b]).wait()


@functools.partial(
    pl.kernel,
    out_type=jax.ShapeDtypeStruct((NC, ACC_ROWS, D_LAT), jnp.float32),
    mesh=_sc_mesh,
    scratch_types=[
        pltpu.VMEM((CH,), jnp.int32),
        pltpu.VMEM((CH,), jnp.int32),
        pltpu.VMEM((CH, D_LAT), jnp.float32),
        pltpu.VMEM((CH, D_LAT), jnp.float32),
        pltpu.VMEM((TAIL,), jnp.int32),
        pltpu.VMEM((TAIL, D_LAT), jnp.float32),
        pltpu.VMEM_SHARED((ACC_ROWS, D_LAT), jnp.float32),
    ] + [pltpu.SemaphoreType.DMA] * 4,
)
def _sc_scatter(e_hbm, dst_hbm, zeros_hbm, out_hbm,
                di0, di1, rows0, rows1, di_t, rows_t, acc,
                sai0, sai1, sar0, sar1):
    c = lax.axis_index("c")
    s = lax.axis_index("s")
    wid = s * NC + c
    base = wid * EPW

    pltpu.sync_copy(zeros_hbm.at[pl.ds(s * NPS, NPS)], acc.at[pl.ds(s * NPS, NPS)])
    plsc.subcore_barrier()

    di = (di0, di1)
    rows = (rows0, rows1)
    sai = (sai0, sai1)
    sar = (sar0, sar1)

    def idx_off(ci):
        return pl.ds(base + ci * CH, CH)

    for b in (0, 1):
        pltpu.async_copy(dst_hbm.at[idx_off(b)], di[b], sai[b])
        pltpu.async_copy(e_hbm.at[idx_off(b)], rows[b], sar[b])

    @pl.loop(0, N_FULL // 2)
    def _(g):
        for b in (0, 1):
            ci = g * 2 + b
            pltpu.make_async_copy(dst_hbm.at[idx_off(ci)], di[b], sai[b]).wait()
            pltpu.make_async_copy(e_hbm.at[idx_off(ci)], rows[b], sar[b]).wait()
            pltpu.sync_copy(rows[b], acc.at[di[b]], add=True)

            @pl.when(ci < N_FULL - 2)
            def _():
                pltpu.async_copy(dst_hbm.at[idx_off(ci + 2)], di[b], sai[b])
                pltpu.async_copy(e_hbm.at[idx_off(ci + 2)], rows[b], sar[b])

    off = pl.ds(base + N_FULL * CH, TAIL)
    pltpu.sync_copy(dst_hbm.at[off], di_t)
    pltpu.sync_copy(e_hbm.at[off], rows_t)
    pltpu.sync_copy(rows_t, acc.at[di_t], add=True)

    plsc.subcore_barrier()
    pltpu.sync_copy(acc.at[pl.ds(s * NPS, NPS)], out_hbm.at[c].at[pl.ds(s * NPS, NPS)])


_R_NODE = 2000
_R_EDGE = 2000


def _wspec(r, c):
    return pl.BlockSpec((r, c), lambda i: (0, 0))


def _rspec(r, c):
    return pl.BlockSpec((r, c), lambda i: (i, 0))


def _node_encode_body(x_ref, w1, b1, w2, b2, ws, wd, v_ref, vs_ref, vd_ref):
    h = _dot(x_ref[...], w1[...]) + b1[...]
    v = _dot(h, w2[...]) + b2[...]
    v_ref[...] = v
    vs_ref[...] = _dot(v, ws[...])
    vd_ref[...] = _dot(v, wd[...])


def _gsum(gs_ref, gd_ref):
    return gs_ref[...].reshape(gs_ref.shape[1:]) + gd_ref[...].reshape(gd_ref.shape[1:])


def _edge_step1_body(ef_ref, gs_ref, gd_ref, we1, be1, we2, be2,
                     w1e, b1, w2, b2, out_ref):
    e0 = _dot(ef_ref[...], we1[...]) + be1[...]
    e0 = _dot(e0, we2[...]) + be2[...]
    g = _gsum(gs_ref, gd_ref)
    h = _dot(e0, w1e[...]) + g + b1[...]
    out_ref[...] = e0 + _dot(h, w2[...]) + b2[...]


def _edge_step2_body(e_ref, gs_ref, gd_ref, w1e, b1, w2, b2, out_ref):
    g = _gsum(gs_ref, gd_ref)
    h = _dot(e_ref[...], w1e[...]) + g + b1[...]
    out_ref[...] = e_ref[...] + _dot(h, w2[...]) + b2[...]


def _node_update_body(v_ref, p0_ref, p1_ref, w1v, w1a, b1, w2, b2, ws, wd,
                      v1_ref, vs_ref, vd_ref):
    agg = p0_ref[...] + p1_ref[...]
    h = _dot(v_ref[...], w1v[...]) + _dot(agg, w1a[...]) + b1[...]
    v1 = v_ref[...] + _dot(h, w2[...]) + b2[...]
    v1_ref[...] = v1
    vs_ref[...] = _dot(v1, ws[...])
    vd_ref[...] = _dot(v1, wd[...])


def _node_final_body(v_ref, p0_ref, p1_ref, w1v, w1a, b1, w2, b2,
                     d1, db1, d2, db2, out_ref):
    agg = p0_ref[...] + p1_ref[...]
    h = _dot(v_ref[...], w1v[...]) + _dot(agg, w1a[...]) + b1[...]
    v2 = v_ref[...] + _dot(h, w2[...]) + b2[...]
    o = _dot(v2, d1[...]) + db1[...]
    out_ref[...] = _dot(o, d2[...]) + db2[...]


def _tc_call(body, grid, in_specs, out_specs, out_shapes, *args):
    return pl.pallas_call(
        body,
        grid=(grid,),
        in_specs=in_specs,
        out_specs=out_specs,
        out_shape=out_shapes,
        compiler_params=pltpu.CompilerParams(
            dimension_semantics=("arbitrary",)),
    )(*args)


def kernel(node_features_in, edges_indexes, edge_features_in, params):
    f32 = jnp.float32
    src = edges_indexes[0]
    dst = edges_indexes[1]

    def _wb(layer):
        return layer["W"], layer["b"].reshape(1, -1)

    enW1, enb1 = _wb(params["enc_node"][0])
    enW2, enb2 = _wb(params["enc_node"][1])
    eeW1, eeb1 = _wb(params["enc_edge"][0])
    eeW2, eeb2 = _wb(params["enc_edge"][1])
    dW1, db1 = _wb(params["dec"][0])
    dW2, db2 = _wb(params["dec"][1])

    steps = []
    for t in range(2):
        pe = params["proc"][t]["edge"]
        pn = params["proc"][t]["node"]
        W1, b1 = _wb(pe[0])
        W2, b2 = _wb(pe[1])
        nW1, nb1 = _wb(pn[0])
        nW2, nb2 = _wb(pn[1])
        steps.append(dict(
            W1e=W1[:D_LAT], W1s=W1[D_LAT:2 * D_LAT], W1d=W1[2 * D_LAT:],
            b1=b1, W2=W2, b2=b2,
            nW1v=nW1[:D_LAT], nW1a=nW1[D_LAT:], nb1=nb1, nW2=nW2, nb2=nb2,
        ))

    zeros_nodes = jnp.zeros((ACC_ROWS, D_LAT), f32)

    nb = N_NODES // _R_NODE
    w128 = _wspec(D_LAT, D_LAT)
    bia = _wspec(1, D_LAT)
    nrow = _rspec(_R_NODE, D_LAT)
    nshape = jax.ShapeDtypeStruct((N_NODES, D_LAT), f32)

    def _tab(x):
        return jnp.pad(x, ((0, ACC_ROWS - N_NODES), (0, 0)))

    v0, vs1, vd1 = _tc_call(
        _node_encode_body, nb,
        [nrow, w128, bia, w128, bia, w128, w128],
        [nrow, nrow, nrow], [nshape, nshape, nshape],
        node_features_in, enW1, enb1, enW2, enb2,
        steps[0]["W1s"], steps[0]["W1d"])

    ei_pad = jnp.pad(edges_indexes, ((0, 0), (0, N_EPAD - N_EDGES)))

    g1 = _sc_gather(_tab(vs1), _tab(vd1), ei_pad)

    eb = N_EDGES // _R_EDGE
    erow = _rspec(_R_EDGE, D_LAT)
    eshape = jax.ShapeDtypeStruct((N_EDGES, D_LAT), f32)
    g0spec = pl.BlockSpec((1, _R_EDGE, D_LAT), lambda i: (0, i, 0))
    g1spec = pl.BlockSpec((1, _R_EDGE, D_LAT), lambda i: (1, i, 0))
    e1 = _tc_call(
        _edge_step1_body, eb,
        [_rspec(_R_EDGE, 16), g0spec, g1spec,
         _wspec(16, D_LAT), bia, w128, bia, w128, bia, w128, bia],
        erow, eshape,
        edge_features_in, g1, g1, eeW1, eeb1, eeW2, eeb2,
        steps[0]["W1e"], steps[0]["b1"], steps[0]["W2"], steps[0]["b2"])

    part1 = _sc_scatter(e1, dst, zeros_nodes)

    v1, vs2, vd2 = _tc_call(
        _node_update_body, nb,
        [nrow, nrow, nrow, w128, w128, bia, w128, bia, w128, w128],
        [nrow, nrow, nrow], [nshape, nshape, nshape],
        v0, part1[0, :N_NODES], part1[1, :N_NODES],
        steps[0]["nW1v"], steps[0]["nW1a"], steps[0]["nb1"],
        steps[0]["nW2"], steps[0]["nb2"],
        steps[1]["W1s"], steps[1]["W1d"])

    g2 = _sc_gather(_tab(vs2), _tab(vd2), ei_pad)

    e2 = _tc_call(
        _edge_step2_body, eb,
        [erow, g0spec, g1spec, w128, bia, w128, bia],
        erow, eshape,
        e1, g2, g2,
        steps[1]["W1e"], steps[1]["b1"], steps[1]["W2"], steps[1]["b2"])

    part2 = _sc_scatter(e2, dst, zeros_nodes)

    out = _tc_call(
        _node_final_body, nb,
        [nrow, nrow, nrow, w128, w128, bia, w128, bia,
         w128, bia, w128, bia],
        nrow, nshape,
        v1, part2[0, :N_NODES], part2[1, :N_NODES],
        steps[1]["nW1v"], steps[1]["nW1a"], steps[1]["nb1"],
        steps[1]["nW2"], steps[1]["nb2"],
        dW1, db1, dW2, db2)

    return out


# speedup vs baseline: 4.1884x; 1.6601x over previous
import functools

import jax
import jax.numpy as jnp
from jax import lax
from jax.experimental import pallas as pl
from jax.experimental.pallas import tpu as pltpu
from jax.experimental.pallas import tpu_sc as plsc

N_NODES = 10000
N_EDGES = 320000
D_LAT = 128

NC = 2
NS = 16
NW = NC * NS
EPW = N_EDGES // NW
CH = 128
N_FULL = EPW // CH
TAIL = EPW - N_FULL * CH
ACC_ROWS = 10240
NPS = ACC_ROWS // NS

def _dot(a, b):
    f32 = jnp.float32
    bf = jnp.bfloat16
    ah = a.astype(bf)
    al = (a - ah.astype(f32)).astype(bf)
    bh = b.astype(bf)
    bl = (b - bh.astype(f32)).astype(bf)

    def d(x, y):
        return jnp.dot(x, y, preferred_element_type=f32)

    return d(ah, bh) + d(ah, bl) + d(al, bh)


_sc_mesh = plsc.VectorSubcoreMesh(core_axis_name="c", subcore_axis_name="s")

_TSL = ACC_ROWS // NS
N_EPAD = 327680
EPC = N_EPAD // NS
NF2 = EPC // CH


@functools.partial(
    pl.kernel,
    out_type=jax.ShapeDtypeStruct((NC, N_EPAD, D_LAT), jnp.float32),
    mesh=_sc_mesh,
    scratch_types=[
        pltpu.VMEM((CH,), jnp.int32),
        pltpu.VMEM((CH,), jnp.int32),
        pltpu.VMEM((CH, D_LAT), jnp.float32),
        pltpu.VMEM((CH, D_LAT), jnp.float32),
        pltpu.VMEM_SHARED((ACC_ROWS, D_LAT), jnp.float32),
    ] + [pltpu.SemaphoreType.DMA] * 6,
)
def _sc_gather(vs_hbm, vd_hbm, idx_hbm, g_hbm,
               i0, i1, r0, r1, tab_sh,
               sa0, sa1, sb0, sb1, sc0, sc1):
    c = lax.axis_index("c")
    s = lax.axis_index("s")
    base = s * EPC

    sl = pl.ds(s * _TSL, _TSL)

    @pl.when(c == 0)
    def _():
        pltpu.sync_copy(vs_hbm.at[sl], tab_sh.at[sl])

    @pl.when(c == 1)
    def _():
        pltpu.sync_copy(vd_hbm.at[sl], tab_sh.at[sl])

    plsc.subcore_barrier()

    idx = (i0, i1)
    rows = (r0, r1)
    sa = (sa0, sa1)
    sb = (sb0, sb1)
    sc = (sc0, sc1)

    def off(ci):
        return pl.ds(base + ci * CH, CH)

    for b in (0, 1):
        pltpu.async_copy(idx_hbm.at[c].at[off(b)], idx[b], sa[b])

    @pl.loop(0, NF2 // 2)
    def _(g):
        for b in (0, 1):
            ci = g * 2 + b
            pltpu.make_async_copy(idx_hbm.at[c].at[off(ci)], idx[b], sa[b]).wait()

            @pl.when(ci >= 2)
            def _():
                pltpu.make_async_copy(rows[b], g_hbm.at[c].at[off(ci - 2)], sc[b]).wait()

            pltpu.async_copy(tab_sh.at[idx[b]], rows[b], sb[b])
            pltpu.make_async_copy(tab_sh.at[idx[b]], rows[b], sb[b]).wait()
            pltpu.async_copy(rows[b], g_hbm.at[c].at[off(ci)], sc[b])

            @pl.when(ci < NF2 - 2)
            def _():
                pltpu.async_copy(idx_hbm.at[c].at[off(ci + 2)], idx[b], sa[b])

    for b in (0, 1):
        ci = NF2 - 2 + b
        pltpu.make_async_copy(rows[b], g_hbm.at[c].at[off(ci)], sc[b]).wait()


@functools.partial(
    pl.kernel,
    out_type=jax.ShapeDtypeStruct((NC, ACC_ROWS, D_LAT), jnp.float32),
    mesh=_sc_mesh,
    scratch_types=[
        pltpu.VMEM((CH,), jnp.int32),
        pltpu.VMEM((CH,), jnp.int32),
        pltpu.VMEM((CH, D_LAT), jnp.float32),
        pltpu.VMEM((CH, D_LAT), jnp.float32),
        pltpu.VMEM((TAIL,), jnp.int32),
        pltpu.VMEM((TAIL, D_LAT), jnp.float32),
        pltpu.VMEM_SHARED((ACC_ROWS, D_LAT), jnp.float32),
    ] + [pltpu.SemaphoreType.DMA] * 4,
)
def _sc_scatter(e_hbm, dst_hbm, zeros_hbm, out_hbm,
                di0, di1, rows0, rows1, di_t, rows_t, acc,
                sai0, sai1, sar0, sar1):
    c = lax.axis_index("c")
    s = lax.axis_index("s")
    wid = s * NC + c
    base = wid * EPW

    pltpu.sync_copy(zeros_hbm.at[pl.ds(s * NPS, NPS)], acc.at[pl.ds(s * NPS, NPS)])
    plsc.subcore_barrier()

    di = (di0, di1)
    rows = (rows0, rows1)
    sai = (sai0, sai1)
    sar = (sar0, sar1)

    def idx_off(ci):
        return pl.ds(base + ci * CH, CH)

    for b in (0, 1):
        pltpu.async_copy(dst_hbm.at[idx_off(b)], di[b], sai[b])
        pltpu.async_copy(e_hbm.at[idx_off(b)], rows[b], sar[b])

    @pl.loop(0, N_FULL // 2)
    def _(g):
        for b in (0, 1):
            ci = g * 2 + b
            pltpu.make_async_copy(dst_hbm.at[idx_off(ci)], di[b], sai[b]).wait()
            pltpu.make_async_copy(e_hbm.at[idx_off(ci)], rows[b], sar[b]).wait()
            pltpu.sync_copy(rows[b], acc.at[di[b]], add=True)

            @pl.when(ci < N_FULL - 2)
            def _():
                pltpu.async_copy(dst_hbm.at[idx_off(ci + 2)], di[b], sai[b])
                pltpu.async_copy(e_hbm.at[idx_off(ci + 2)], rows[b], sar[b])

    off = pl.ds(base + N_FULL * CH, TAIL)
    pltpu.sync_copy(dst_hbm.at[off], di_t)
    pltpu.sync_copy(e_hbm.at[off], rows_t)
    pltpu.sync_copy(rows_t, acc.at[di_t], add=True)

    plsc.subcore_barrier()
    pltpu.sync_copy(acc.at[pl.ds(s * NPS, NPS)], out_hbm.at[c].at[pl.ds(s * NPS, NPS)])


_R_NODE = 2000
_R_EDGE = 4000


def _wspec(r, c):
    return pl.BlockSpec((r, c), lambda i: (0, 0))


def _rspec(r, c):
    return pl.BlockSpec((r, c), lambda i: (i, 0))


def _node_encode_body(x_ref, w1, b1, w2, b2, ws, wd, v_ref, vs_ref, vd_ref):
    h = _dot(x_ref[...], w1[...]) + b1[...]
    v = _dot(h, w2[...]) + b2[...]
    v_ref[...] = v
    vs_ref[...] = _dot(v, ws[...])
    vd_ref[...] = _dot(v, wd[...])


def _gsum(gs_ref, gd_ref):
    return gs_ref[...].reshape(gs_ref.shape[1:]) + gd_ref[...].reshape(gd_ref.shape[1:])


def _edge_step1_body(ef_ref, gs_ref, gd_ref, we1, be1, we2, be2,
                     w1e, b1, w2, b2, out_ref):
    e0 = _dot(ef_ref[...], we1[...]) + be1[...]
    e0 = _dot(e0, we2[...]) + be2[...]
    g = _gsum(gs_ref, gd_ref)
    h = _dot(e0, w1e[...]) + g + b1[...]
    out_ref[...] = e0 + _dot(h, w2[...]) + b2[...]


def _edge_step2_body(e_ref, gs_ref, gd_ref, w1e, b1, w2, b2, out_ref):
    g = _gsum(gs_ref, gd_ref)
    h = _dot(e_ref[...], w1e[...]) + g + b1[...]
    out_ref[...] = e_ref[...] + _dot(h, w2[...]) + b2[...]


def _node_update_body(v_ref, p0_ref, p1_ref, w1v, w1a, b1, w2, b2, ws, wd,
                      v1_ref, vs_ref, vd_ref):
    agg = p0_ref[...] + p1_ref[...]
    h = _dot(v_ref[...], w1v[...]) + _dot(agg, w1a[...]) + b1[...]
    v1 = v_ref[...] + _dot(h, w2[...]) + b2[...]
    v1_ref[...] = v1
    vs_ref[...] = _dot(v1, ws[...])
    vd_ref[...] = _dot(v1, wd[...])


def _node_final_body(v_ref, p0_ref, p1_ref, w1v, w1a, b1, w2, b2,
                     d1, db1, d2, db2, out_ref):
    agg = p0_ref[...] + p1_ref[...]
    h = _dot(v_ref[...], w1v[...]) + _dot(agg, w1a[...]) + b1[...]
    v2 = v_ref[...] + _dot(h, w2[...]) + b2[...]
    o = _dot(v2, d1[...]) + db1[...]
    out_ref[...] = _dot(o, d2[...]) + db2[...]


def _tc_call(body, grid, in_specs, out_specs, out_shapes, *args):
    return pl.pallas_call(
        body,
        grid=(grid,),
        in_specs=in_specs,
        out_specs=out_specs,
        out_shape=out_shapes,
        compiler_params=pltpu.CompilerParams(
            dimension_semantics=("arbitrary",)),
    )(*args)


def kernel(node_features_in, edges_indexes, edge_features_in, params):
    f32 = jnp.float32
    src = edges_indexes[0]
    dst = edges_indexes[1]

    def _wb(layer):
        return layer["W"], layer["b"].reshape(1, -1)

    enW1, enb1 = _wb(params["enc_node"][0])
    enW2, enb2 = _wb(params["enc_node"][1])
    eeW1, eeb1 = _wb(params["enc_edge"][0])
    eeW2, eeb2 = _wb(params["enc_edge"][1])
    dW1, db1 = _wb(params["dec"][0])
    dW2, db2 = _wb(params["dec"][1])

    steps = []
    for t in range(2):
        pe = params["proc"][t]["edge"]
        pn = params["proc"][t]["node"]
        W1, b1 = _wb(pe[0])
        W2, b2 = _wb(pe[1])
        nW1, nb1 = _wb(pn[0])
        nW2, nb2 = _wb(pn[1])
        steps.append(dict(
            W1e=W1[:D_LAT], W1s=W1[D_LAT:2 * D_LAT], W1d=W1[2 * D_LAT:],
            b1=b1, W2=W2, b2=b2,
            nW1v=nW1[:D_LAT], nW1a=nW1[D_LAT:], nb1=nb1, nW2=nW2, nb2=nb2,
        ))

    zeros_nodes = jnp.zeros((ACC_ROWS, D_LAT), f32)

    nb = N_NODES // _R_NODE
    w128 = _wspec(D_LAT, D_LAT)
    bia = _wspec(1, D_LAT)
    nrow = _rspec(_R_NODE, D_LAT)
    nshape = jax.ShapeDtypeStruct((N_NODES, D_LAT), f32)

    def _tab(x):
        return jnp.pad(x, ((0, ACC_ROWS - N_NODES), (0, 0)))

    v0, vs1, vd1 = _tc_call(
        _node_encode_body, nb,
        [nrow, w128, bia, w128, bia, w128, w128],
        [nrow, nrow, nrow], [nshape, nshape, nshape],
        node_features_in, enW1, enb1, enW2, enb2,
        steps[0]["W1s"], steps[0]["W1d"])

    ei_pad = jnp.pad(edges_indexes, ((0, 0), (0, N_EPAD - N_EDGES)))

    g1 = _sc_gather(_tab(vs1), _tab(vd1), ei_pad)

    eb = N_EDGES // _R_EDGE
    erow = _rspec(_R_EDGE, D_LAT)
    eshape = jax.ShapeDtypeStruct((N_EDGES, D_LAT), f32)
    g0spec = pl.BlockSpec((1, _R_EDGE, D_LAT), lambda i: (0, i, 0))
    g1spec = pl.BlockSpec((1, _R_EDGE, D_LAT), lambda i: (1, i, 0))
    e1 = _tc_call(
        _edge_step1_body, eb,
        [_rspec(_R_EDGE, 16), g0spec, g1spec,
         _wspec(16, D_LAT), bia, w128, bia, w128, bia, w128, bia],
        erow, eshape,
        edge_features_in, g1, g1, eeW1, eeb1, eeW2, eeb2,
        steps[0]["W1e"], steps[0]["b1"], steps[0]["W2"], steps[0]["b2"])

    part1 = _sc_scatter(e1, dst, zeros_nodes)

    v1, vs2, vd2 = _tc_call(
        _node_update_body, nb,
        [nrow, nrow, nrow, w128, w128, bia, w128, bia, w128, w128],
        [nrow, nrow, nrow], [nshape, nshape, nshape],
        v0, part1[0, :N_NODES], part1[1, :N_NODES],
        steps[0]["nW1v"], steps[0]["nW1a"], steps[0]["nb1"],
        steps[0]["nW2"], steps[0]["nb2"],
        steps[1]["W1s"], steps[1]["W1d"])

    g2 = _sc_gather(_tab(vs2), _tab(vd2), ei_pad)

    e2 = _tc_call(
        _edge_step2_body, eb,
        [erow, g0spec, g1spec, w128, bia, w128, bia],
        erow, eshape,
        e1, g2, g2,
        steps[1]["W1e"], steps[1]["b1"], steps[1]["W2"], steps[1]["b2"])

    part2 = _sc_scatter(e2, dst, zeros_nodes)

    out = _tc_call(
        _node_final_body, nb,
        [nrow, nrow, nrow, w128, w128, bia, w128, bia,
         w128, bia, w128, bia],
        nrow, nshape,
        v1, part2[0, :N_NODES], part2[1, :N_NODES],
        steps[1]["nW1v"], steps[1]["nW1a"], steps[1]["nb1"],
        steps[1]["nW2"], steps[1]["nb2"],
        dW1, db1, dW2, db2)

    return out


# speedup vs baseline: 4.3569x; 1.0402x over previous
import functools

import jax
import jax.numpy as jnp
from jax import lax
from jax.experimental import pallas as pl
from jax.experimental.pallas import tpu as pltpu
from jax.experimental.pallas import tpu_sc as plsc

N_NODES = 10000
N_EDGES = 320000
D_LAT = 128

NC = 2
NS = 16
NW = NC * NS
CH = 128
ACC_ROWS = 10240
NPS = ACC_ROWS // NS
_TSL = ACC_ROWS // NS
N_EPAD = 327680
HALF = N_EPAD // 2


def _dot(a, b):
    f32 = jnp.float32
    bf = jnp.bfloat16
    ah = a.astype(bf)
    al = (a - ah.astype(f32)).astype(bf)
    bh = b.astype(bf)
    bl = (b - bh.astype(f32)).astype(bf)

    def d(x, y):
        return jnp.dot(x, y, preferred_element_type=f32)

    return d(ah, bh) + d(ah, bl) + d(al, bh)


_sc_mesh = plsc.VectorSubcoreMesh(core_axis_name="c", subcore_axis_name="s")


def _make_gather(nrows):
    epc = nrows // NS
    nf = epc // CH
    assert nf % 2 == 0 and nf * CH == epc

    @functools.partial(
        pl.kernel,
        out_type=jax.ShapeDtypeStruct((NC, nrows, D_LAT), jnp.float32),
        mesh=_sc_mesh,
        scratch_types=[
            pltpu.VMEM((CH,), jnp.int32),
            pltpu.VMEM((CH,), jnp.int32),
            pltpu.VMEM((CH, D_LAT), jnp.float32),
            pltpu.VMEM((CH, D_LAT), jnp.float32),
            pltpu.VMEM_SHARED((ACC_ROWS, D_LAT), jnp.float32),
        ] + [pltpu.SemaphoreType.DMA] * 6,
    )
    def gather(vs_hbm, vd_hbm, idx_hbm, g_hbm,
               i0, i1, r0, r1, tab_sh,
               sa0, sa1, sb0, sb1, sc0, sc1):
        c = lax.axis_index("c")
        s = lax.axis_index("s")
        base = s * epc

        sl = pl.ds(s * _TSL, _TSL)

        @pl.when(c == 0)
        def _():
            pltpu.sync_copy(vs_hbm.at[sl], tab_sh.at[sl])

        @pl.when(c == 1)
        def _():
            pltpu.sync_copy(vd_hbm.at[sl], tab_sh.at[sl])

        plsc.subcore_barrier()

        idx = (i0, i1)
        rows = (r0, r1)
        sa = (sa0, sa1)
        sb = (sb0, sb1)
        sc = (sc0, sc1)

        def off(ci):
            return pl.ds(base + ci * CH, CH)

        for b in (0, 1):
            pltpu.async_copy(idx_hbm.at[c].at[off(b)], idx[b], sa[b])

        @pl.loop(0, nf // 2)
        def _(g):
            for b in (0, 1):
                ci = g * 2 + b
                pltpu.make_async_copy(idx_hbm.at[c].at[off(ci)], idx[b], sa[b]).wait()

                @pl.when(ci >= 2)
                def _():
                    pltpu.make_async_copy(rows[b], g_hbm.at[c].at[off(ci - 2)], sc[b]).wait()

                pltpu.async_copy(tab_sh.at[idx[b]], rows[b], sb[b])
                pltpu.make_async_copy(tab_sh.at[idx[b]], rows[b], sb[b]).wait()
                pltpu.async_copy(rows[b], g_hbm.at[c].at[off(ci)], sc[b])

                @pl.when(ci < nf - 2)
                def _():
                    pltpu.async_copy(idx_hbm.at[c].at[off(ci + 2)], idx[b], sa[b])

        for b in (0, 1):
            ci = nf - 2 + b
            pltpu.make_async_copy(rows[b], g_hbm.at[c].at[off(ci)], sc[b]).wait()

    return gather


def _make_scatter(nrows):
    epw = nrows // NW
    nf = epw // CH
    assert nf % 2 == 0 and nf * CH == epw

    @functools.partial(
        pl.kernel,
        out_type=jax.ShapeDtypeStruct((NC, ACC_ROWS, D_LAT), jnp.float32),
        mesh=_sc_mesh,
        scratch_types=[
            pltpu.VMEM((CH,), jnp.int32),
            pltpu.VMEM((CH,), jnp.int32),
            pltpu.VMEM((CH, D_LAT), jnp.float32),
            pltpu.VMEM((CH, D_LAT), jnp.float32),
            pltpu.VMEM_SHARED((ACC_ROWS, D_LAT), jnp.float32),
        ] + [pltpu.SemaphoreType.DMA] * 4,
    )
    def scatter(e_hbm, dst_hbm, zeros_hbm, out_hbm,
                di0, di1, rows0, rows1, acc,
                sai0, sai1, sar0, sar1):
        c = lax.axis_index("c")
        s = lax.axis_index("s")
        wid = s * NC + c
        base = wid * epw

        pltpu.sync_copy(zeros_hbm.at[pl.ds(s * NPS, NPS)], acc.at[pl.ds(s * NPS, NPS)])
        plsc.subcore_barrier()

        di = (di0, di1)
        rows = (rows0, rows1)
        sai = (sai0, sai1)
        sar = (sar0, sar1)

        def off(ci):
            return pl.ds(base + ci * CH, CH)

        for b in (0, 1):
            pltpu.async_copy(dst_hbm.at[off(b)], di[b], sai[b])
            pltpu.async_copy(e_hbm.at[off(b)], rows[b], sar[b])

        @pl.loop(0, nf // 2)
        def _(g):
            for b in (0, 1):
                ci = g * 2 + b
                pltpu.make_async_copy(dst_hbm.at[off(ci)], di[b], sai[b]).wait()
                pltpu.make_async_copy(e_hbm.at[off(ci)], rows[b], sar[b]).wait()
                pltpu.sync_copy(rows[b], acc.at[di[b]], add=True)

                @pl.when(ci < nf - 2)
                def _():
                    pltpu.async_copy(dst_hbm.at[off(ci + 2)], di[b], sai[b])
                    pltpu.async_copy(e_hbm.at[off(ci + 2)], rows[b], sar[b])

        plsc.subcore_barrier()
        pltpu.sync_copy(acc.at[pl.ds(s * NPS, NPS)], out_hbm.at[c].at[pl.ds(s * NPS, NPS)])

    return scatter


_gather_half = _make_gather(HALF)
_scatter_half = _make_scatter(HALF)


_R_NODE = 2000
_R_EDGE = 8192


def _wspec(r, c):
    return pl.BlockSpec((r, c), lambda i: (0, 0))


def _rspec(r, c):
    return pl.BlockSpec((r, c), lambda i: (i, 0))


def _node_encode_body(x_ref, w1, b1, w2, b2, ws, wd, v_ref, vs_ref, vd_ref):
    h = _dot(x_ref[...], w1[...]) + b1[...]
    v = _dot(h, w2[...]) + b2[...]
    v_ref[...] = v
    vs_ref[...] = _dot(v, ws[...])
    vd_ref[...] = _dot(v, wd[...])


def _gsum(gs_ref, gd_ref):
    return gs_ref[...].reshape(gs_ref.shape[1:]) + gd_ref[...].reshape(gd_ref.shape[1:])


def _edge_step1_body(ef_ref, gs_ref, gd_ref, we1, be1, we2, be2,
                     w1e, b1, w2, b2, out_ref):
    e0 = _dot(ef_ref[...], we1[...]) + be1[...]
    e0 = _dot(e0, we2[...]) + be2[...]
    g = _gsum(gs_ref, gd_ref)
    h = _dot(e0, w1e[...]) + g + b1[...]
    out_ref[...] = e0 + _dot(h, w2[...]) + b2[...]


def _edge_step2_body(e_ref, gs_ref, gd_ref, w1e, b1, w2, b2, out_ref):
    g = _gsum(gs_ref, gd_ref)
    h = _dot(e_ref[...], w1e[...]) + g + b1[...]
    out_ref[...] = e_ref[...] + _dot(h, w2[...]) + b2[...]


def _agg4(p0_ref, p1_ref, p2_ref, p3_ref):
    return p0_ref[...] + p1_ref[...] + p2_ref[...] + p3_ref[...]


def _node_update_body(v_ref, p0, p1, p2, p3, w1v, w1a, b1, w2, b2, ws, wd,
                      v1_ref, vs_ref, vd_ref):
    agg = _agg4(p0, p1, p2, p3)
    h = _dot(v_ref[...], w1v[...]) + _dot(agg, w1a[...]) + b1[...]
    v1 = v_ref[...] + _dot(h, w2[...]) + b2[...]
    v1_ref[...] = v1
    vs_ref[...] = _dot(v1, ws[...])
    vd_ref[...] = _dot(v1, wd[...])


def _node_final_body(v_ref, p0, p1, p2, p3, w1v, w1a, b1, w2, b2,
                     d1, db1, d2, db2, out_ref):
    agg = _agg4(p0, p1, p2, p3)
    h = _dot(v_ref[...], w1v[...]) + _dot(agg, w1a[...]) + b1[...]
    v2 = v_ref[...] + _dot(h, w2[...]) + b2[...]
    o = _dot(v2, d1[...]) + db1[...]
    out_ref[...] = _dot(o, d2[...]) + db2[...]


def _tc_call(body, grid, in_specs, out_specs, out_shapes, *args):
    return pl.pallas_call(
        body,
        grid=(grid,),
        in_specs=in_specs,
        out_specs=out_specs,
        out_shape=out_shapes,
        compiler_params=pltpu.CompilerParams(
            dimension_semantics=("arbitrary",)),
    )(*args)


def kernel(node_features_in, edges_indexes, edge_features_in, params):
    f32 = jnp.float32
    npad = N_EPAD - N_EDGES
    src_pad = jnp.concatenate(
        [edges_indexes[0], jnp.zeros((npad,), jnp.int32)])
    dst_pad = jnp.concatenate(
        [edges_indexes[1],
         N_NODES + (jnp.arange(npad, dtype=jnp.int32) % (ACC_ROWS - N_NODES))])
    ei = jnp.stack([src_pad, dst_pad])
    ei_a, ei_b = ei[:, :HALF], ei[:, HALF:]
    dst_a, dst_b = dst_pad[:HALF], dst_pad[HALF:]
    ef_pad = jnp.pad(edge_features_in, ((0, npad), (0, 0)))
    ef_a, ef_b = ef_pad[:HALF], ef_pad[HALF:]

    def _wb(layer):
        return layer["W"], layer["b"].reshape(1, -1)

    enW1, enb1 = _wb(params["enc_node"][0])
    enW2, enb2 = _wb(params["enc_node"][1])
    eeW1, eeb1 = _wb(params["enc_edge"][0])
    eeW2, eeb2 = _wb(params["enc_edge"][1])
    dW1, db1 = _wb(params["dec"][0])
    dW2, db2 = _wb(params["dec"][1])

    steps = []
    for t in range(2):
        pe = params["proc"][t]["edge"]
        pn = params["proc"][t]["node"]
        W1, b1 = _wb(pe[0])
        W2, b2 = _wb(pe[1])
        nW1, nb1 = _wb(pn[0])
        nW2, nb2 = _wb(pn[1])
        steps.append(dict(
            W1e=W1[:D_LAT], W1s=W1[D_LAT:2 * D_LAT], W1d=W1[2 * D_LAT:],
            b1=b1, W2=W2, b2=b2,
            nW1v=nW1[:D_LAT], nW1a=nW1[D_LAT:], nb1=nb1, nW2=nW2, nb2=nb2,
        ))

    zeros_nodes = jnp.zeros((ACC_ROWS, D_LAT), f32)

    nb = N_NODES // _R_NODE
    w128 = _wspec(D_LAT, D_LAT)
    bia = _wspec(1, D_LAT)
    nrow = _rspec(_R_NODE, D_LAT)
    nshape = jax.ShapeDtypeStruct((N_NODES, D_LAT), f32)

    eb = HALF // _R_EDGE
    erow = _rspec(_R_EDGE, D_LAT)
    eshape = jax.ShapeDtypeStruct((HALF, D_LAT), f32)
    g0spec = pl.BlockSpec((1, _R_EDGE, D_LAT), lambda i: (0, i, 0))
    g1spec = pl.BlockSpec((1, _R_EDGE, D_LAT), lambda i: (1, i, 0))

    def _tab(x):
        return jnp.pad(x, ((0, ACC_ROWS - N_NODES), (0, 0)))

    def edge_step1(ef_h, g_h, st):
        return _tc_call(
            _edge_step1_body, eb,
            [_rspec(_R_EDGE, 16), g0spec, g1spec,
             _wspec(16, D_LAT), bia, w128, bia, w128, bia, w128, bia],
            erow, eshape,
            ef_h, g_h, g_h, eeW1, eeb1, eeW2, eeb2,
            st["W1e"], st["b1"], st["W2"], st["b2"])

    def edge_step2(e_h, g_h, st):
        return _tc_call(
            _edge_step2_body, eb,
            [erow, g0spec, g1spec, w128, bia, w128, bia],
            erow, eshape,
            e_h, g_h, g_h,
            st["W1e"], st["b1"], st["W2"], st["b2"])

    v0, vs1, vd1 = _tc_call(
        _node_encode_body, nb,
        [nrow, w128, bia, w128, bia, w128, w128],
        [nrow, nrow, nrow], [nshape, nshape, nshape],
        node_features_in, enW1, enb1, enW2, enb2,
        steps[0]["W1s"], steps[0]["W1d"])

    t1s, t1d = _tab(vs1), _tab(vd1)
    g1a = _gather_half(t1s, t1d, ei_a)
    g1b = _gather_half(t1s, t1d, ei_b)
    e1a = edge_step1(ef_a, g1a, steps[0])
    e1b = edge_step1(ef_b, g1b, steps[0])
    p1a = _scatter_half(e1a, dst_a, zeros_nodes)
    p1b = _scatter_half(e1b, dst_b, zeros_nodes)

    v1, vs2, vd2 = _tc_call(
        _node_update_body, nb,
        [nrow, nrow, nrow, nrow, nrow, w128, w128, bia, w128, bia, w128, w128],
        [nrow, nrow, nrow], [nshape, nshape, nshape],
        v0, p1a[0, :N_NODES], p1a[1, :N_NODES],
        p1b[0, :N_NODES], p1b[1, :N_NODES],
        steps[0]["nW1v"], steps[0]["nW1a"], steps[0]["nb1"],
        steps[0]["nW2"], steps[0]["nb2"],
        steps[1]["W1s"], steps[1]["W1d"])

    t2s, t2d = _tab(vs2), _tab(vd2)
    g2a = _gather_half(t2s, t2d, ei_a)
    g2b = _gather_half(t2s, t2d, ei_b)
    e2a = edge_step2(e1a, g2a, steps[1])
    e2b = edge_step2(e1b, g2b, steps[1])
    p2a = _scatter_half(e2a, dst_a, zeros_nodes)
    p2b = _scatter_half(e2b, dst_b, zeros_nodes)

    out = _tc_call(
        _node_final_body, nb,
        [nrow, nrow, nrow, nrow, nrow, w128, w128, bia, w128, bia,
         w128, bia, w128, bia],
        nrow, nshape,
        v1, p2a[0, :N_NODES], p2a[1, :N_NODES],
        p2b[0, :N_NODES], p2b[1, :N_NODES],
        steps[1]["nW1v"], steps[1]["nW1a"], steps[1]["nb1"],
        steps[1]["nW2"], steps[1]["nb2"],
        dW1, db1, dW2, db2)

    return out
